# baseline (speedup 1.0000x reference)
import functools

import jax
import jax.numpy as jnp
from jax import lax
from jax.experimental import pallas as pl
from jax.experimental.pallas import tpu as pltpu
from jax.experimental.pallas import tpu_sc as plsc

B, C, I, N = 256, 64, 64, 32
ROW = N * 2
NW = 32
B_PER_W = B // NW
L = 16
CB = 4
IV = I // L
NI = N * I
IR = I * ROW


def _matches_and_gather(xy_hbm, inst_hbm, cent_hbm, out_hbm,
                        xy_v, inst_v, cent_v, rows_v,
                        sem_s, sem_o):
    wid = lax.axis_index("s") * 2 + lax.axis_index("c")
    b0 = wid * B_PER_W
    lanes = lax.iota(jnp.int32, L)
    inf_l = jnp.full((L,), jnp.inf, jnp.float32)
    jbase = [jnp.full((L,), 16 * j, jnp.int32) for j in range(IV)]
    kbase = [jnp.full((L,), 16 * k, jnp.int32) + lanes for k in range(4)]

    def stage_copies(bi, buf):
        b = b0 + bi
        return (
            pltpu.make_async_copy(
                xy_hbm.at[pl.ds(b * 2 * NI, 2 * NI)],
                xy_v.at[pl.ds(buf * 2 * NI, 2 * NI)], sem_s),
            pltpu.make_async_copy(
                inst_hbm.at[pl.ds(b * IR, IR)],
                inst_v.at[pl.ds(buf * IR, IR)], sem_s),
            pltpu.make_async_copy(
                cent_hbm.at[pl.ds(b * C * 2, C * 2)],
                cent_v.at[pl.ds(buf * C * 2, C * 2)], sem_s),
        )

    def out_copy(bi):
        buf = bi % 2
        return pltpu.make_async_copy(
            rows_v.at[pl.ds(buf * C * ROW, C * ROW)],
            out_hbm.at[pl.ds((b0 + bi) * C * ROW, C * ROW)], sem_o)

    for c_ in stage_copies(0, 0):
        c_.start()

    def batch_body(bi, _):
        buf = bi % 2
        for c_ in stage_copies(bi, buf):
            c_.wait()

        @pl.when(bi + 1 < B_PER_W)
        def _():
            for c_ in stage_copies(bi + 1, 1 - buf):
                c_.start()

        @pl.when(bi >= 2)
        def _():
            out_copy(bi - 2).wait()

        xoff = buf * 2 * NI
        yoff = xoff + NI
        coff = buf * C * 2
        ioff = jnp.full((L,), buf * IR, jnp.int32)
        roff = buf * C * ROW

        for jj in range(C // L):
            cxs = cent_v[pl.ds(coff + L * jj, L)]
            cys = cent_v[pl.ds(coff + C + L * jj, L)]

            def cc_body(cc, match16, cxs=cxs, cys=cys, jj=jj):
                cxv = [jnp.take_along_axis(
                    cxs, jnp.full((L,), CB * cc + u, jnp.int32), axis=0)
                    for u in range(CB)]
                cyv = [jnp.take_along_axis(
                    cys, jnp.full((L,), CB * cc + u, jnp.int32), axis=0)
                    for u in range(CB)]

                dmin = [inf_l] * (CB * IV)
                for n in range(N):
                    for j in range(IV):
                        ax = xy_v[pl.ds(xoff + n * I + L * j, L)]
                        ay = xy_v[pl.ds(yoff + n * I + L * j, L)]
                        for u in range(CB):
                            dx = ax - cxv[u]
                            dy = ay - cyv[u]
                            d = dx * dx + dy * dy
                            k = u * IV + j
                            dmin[k] = jnp.minimum(dmin[k], d)

                for u in range(CB):
                    v = dmin[u * IV]
                    iid = jbase[0] + lanes
                    for j in range(1, IV):
                        dj = dmin[u * IV + j]
                        m = dj < v
                        v = jnp.where(m, dj, v)
                        iid = jnp.where(m, jbase[j] + lanes, iid)
                    for st in (1, 2, 4, 8):
                        pv = jnp.take_along_axis(v, lanes ^ st, axis=0)
                        pid = jnp.take_along_axis(iid, lanes ^ st, axis=0)
                        m = (pv < v) | ((pv == v) & (pid < iid))
                        v = jnp.where(m, pv, v)
                        iid = jnp.where(m, pid, iid)
                    match16 = jnp.where(
                        lanes == jnp.full((L,), CB * cc + u, jnp.int32),
                        iid, match16)
                return match16

            match16 = lax.fori_loop(0, CB, cc_body,
                                    jnp.zeros((L,), jnp.int32))

            for c16 in range(L):
                mc = jnp.take_along_axis(
                    match16, jnp.full((L,), c16, jnp.int32), axis=0)
                base = ioff + mc * ROW
                for k in range(4):
                    rv = plsc.load_gather(inst_v, [base + kbase[k]])
                    rows_v[pl.ds(roff + (jj * L + c16) * ROW + L * k, L)] = rv

        out_copy(bi).start()
        return 0

    lax.fori_loop(0, B_PER_W, batch_body, 0)

    @pl.when(B_PER_W >= 2)
    def _():
        out_copy(B_PER_W - 2).wait()
    out_copy(B_PER_W - 1).wait()


@jax.jit
def kernel(instances, centroids, centroid_vals):
    xy = instances.transpose(0, 3, 2, 1).reshape(B * 2 * NI)
    cent_t = centroids.transpose(0, 2, 1).reshape(B * 2 * C)
    inst_1d = instances.reshape(B * IR)
    mesh = plsc.VectorSubcoreMesh(core_axis_name="c", subcore_axis_name="s")
    k = functools.partial(
        pl.kernel,
        mesh=mesh,
        compiler_params=pltpu.CompilerParams(
            needs_layout_passes=False, use_tc_tiling_on_sc=False),
        out_type=jax.ShapeDtypeStruct((B * C * ROW,), jnp.float32),
        scratch_types=[
            pltpu.VMEM((2 * 2 * NI,), jnp.float32),
            pltpu.VMEM((2 * IR,), jnp.float32),
            pltpu.VMEM((2 * C * 2,), jnp.float32),
            pltpu.VMEM((2 * C * ROW,), jnp.float32),
            pltpu.SemaphoreType.DMA,
            pltpu.SemaphoreType.DMA,
        ],
    )(_matches_and_gather)
    peaks = k(xy, inst_1d, cent_t)
    instance_peaks = peaks.reshape(B, C, N, 2)
    instance_peak_vals = jnp.ones((B, C, N), jnp.float32)
    return (centroids, centroid_vals, instance_peaks, instance_peak_vals)


# speedup vs baseline: 4.4303x; 4.4303x over previous
import functools

import jax
import jax.numpy as jnp
from jax import lax
from jax.experimental import pallas as pl
from jax.experimental.pallas import tpu as pltpu
from jax.experimental.pallas import tpu_sc as plsc

B, C, I, N = 256, 64, 64, 32
ROW = N * 2
NW = 32
B_PER_W = B // NW
L = 16
CB = 4
IV = I // L
NI = N * I


def _matches_and_gather(xy_hbm, inst_hbm, cent_hbm, out_hbm,
                        xy_v, cent_v, idx_v, rows_v,
                        sem_s, sem_g, sem_o):
    wid = lax.axis_index("s") * 2 + lax.axis_index("c")
    b0 = wid * B_PER_W
    lanes = lax.iota(jnp.int32, L)
    inf_l = jnp.full((L,), jnp.inf, jnp.float32)
    jbase = [jnp.full((L,), 16 * j, jnp.int32) for j in range(IV)]

    def stage_copies(bi, buf):
        b = b0 + bi
        return (
            pltpu.make_async_copy(
                xy_hbm.at[pl.ds(b * 2 * NI, 2 * NI)],
                xy_v.at[pl.ds(buf * 2 * NI, 2 * NI)], sem_s),
            pltpu.make_async_copy(
                cent_hbm.at[pl.ds(b * C * 2, C * 2)],
                cent_v.at[pl.ds(buf * C * 2, C * 2)], sem_s),
        )

    def gather_copy(bi):
        return pltpu.make_async_copy(
            inst_hbm.at[idx_v.at[pl.ds(bi * C, C)]],
            rows_v.at[pl.ds(bi * C, C)], sem_g)

    def out_copy(bi):
        return pltpu.make_async_copy(
            rows_v.at[pl.ds(bi * C, C)],
            out_hbm.at[pl.ds((b0 + bi) * C, C)], sem_o)

    for c_ in stage_copies(0, 0):
        c_.start()

    def batch_body(bi, _):
        buf = bi % 2
        b = b0 + bi
        for c_ in stage_copies(bi, buf):
            c_.wait()

        @pl.when(bi + 1 < B_PER_W)
        def _():
            for c_ in stage_copies(bi + 1, 1 - buf):
                c_.start()

        b64 = jnp.full((L,), b * I, jnp.int32)
        xoff = buf * 2 * NI
        yoff = xoff + NI
        coff = buf * C * 2

        for jj in range(C // L):
            cxs = cent_v[pl.ds(coff + L * jj, L)]
            cys = cent_v[pl.ds(coff + C + L * jj, L)]

            def cc_body(cc, _, cxs=cxs, cys=cys, jj=jj):
                cxv = [jnp.take_along_axis(
                    cxs, jnp.full((L,), CB * cc + u, jnp.int32), axis=0)
                    for u in range(CB)]
                cyv = [jnp.take_along_axis(
                    cys, jnp.full((L,), CB * cc + u, jnp.int32), axis=0)
                    for u in range(CB)]

                dmin = [inf_l] * (CB * IV)
                for n in range(N):
                    for j in range(IV):
                        ax = xy_v[pl.ds(xoff + n * I + L * j, L)]
                        ay = xy_v[pl.ds(yoff + n * I + L * j, L)]
                        for u in range(CB):
                            dx = ax - cxv[u]
                            dy = ay - cyv[u]
                            d = dx * dx + dy * dy
                            k = u * IV + j
                            dmin[k] = jnp.minimum(dmin[k], d)

                for u in range(CB):
                    v = dmin[u * IV]
                    iid = jbase[0] + lanes
                    for j in range(1, IV):
                        dj = dmin[u * IV + j]
                        m = dj < v
                        v = jnp.where(m, dj, v)
                        iid = jnp.where(m, jbase[j] + lanes, iid)
                    for st in (1, 2, 4, 8):
                        pv = jnp.take_along_axis(v, lanes ^ st, axis=0)
                        pid = jnp.take_along_axis(iid, lanes ^ st, axis=0)
                        m = (pv < v) | ((pv == v) & (pid < iid))
                        v = jnp.where(m, pv, v)
                        iid = jnp.where(m, pid, iid)
                    cpos = jnp.full(
                        (L,), bi * C + L * jj + CB * cc + u, jnp.int32)
                    plsc.store_scatter(
                        idx_v, [cpos], iid + b64, mask=lanes == 0)
                return 0

            lax.fori_loop(0, CB, cc_body, 0)

        @pl.when(bi >= 1)
        def _():
            gather_copy(bi - 1).wait()
            out_copy(bi - 1).start()

        gather_copy(bi).start()
        return 0

    lax.fori_loop(0, B_PER_W, batch_body, 0)

    gather_copy(B_PER_W - 1).wait()
    out_copy(B_PER_W - 1).start()
    for bi in range(B_PER_W):
        out_copy(bi).wait()


@jax.jit
def kernel(instances, centroids, centroid_vals):
    xy = instances.transpose(0, 3, 2, 1).reshape(B * 2 * NI)
    cent_t = centroids.transpose(0, 2, 1).reshape(B * 2 * C)
    inst_rows = instances.reshape(B * I, ROW)
    mesh = plsc.VectorSubcoreMesh(core_axis_name="c", subcore_axis_name="s")
    k = functools.partial(
        pl.kernel,
        mesh=mesh,
        compiler_params=pltpu.CompilerParams(
            needs_layout_passes=False, use_tc_tiling_on_sc=False),
        out_type=jax.ShapeDtypeStruct((B * C, ROW), jnp.float32),
        scratch_types=[
            pltpu.VMEM((2 * 2 * NI,), jnp.float32),
            pltpu.VMEM((2 * C * 2,), jnp.float32),
            pltpu.VMEM((B_PER_W * C,), jnp.int32),
            pltpu.VMEM((B_PER_W * C, ROW), jnp.float32),
            pltpu.SemaphoreType.DMA,
            pltpu.SemaphoreType.DMA,
            pltpu.SemaphoreType.DMA,
        ],
    )(_matches_and_gather)
    peaks = k(xy, inst_rows, cent_t)
    instance_peaks = peaks.reshape(B, C, N, 2)
    instance_peak_vals = jnp.ones((B, C, N), jnp.float32)
    return (centroids, centroid_vals, instance_peaks, instance_peak_vals)
